# baseline (speedup 1.0000x reference)
import functools

import jax
import jax.numpy as jnp
from jax import lax
from jax.experimental import pallas as pl
from jax.experimental.pallas import tpu as pltpu
from jax.experimental.pallas import tpu_sc as plsc

D = 128
B = 4096
L = 50
LPAD = 56
NW = 32
SEQ_PER_W = B // NW
CHUNK_SEQ = 4
NCHUNK = SEQ_PER_W // CHUNK_SEQ
NVEC = D // 16
NBUF = 4


def _issue_chunk(x_hbm, tok_hbm, idx_v, rows_v, gsem, kb, b0):
    pltpu.sync_copy(x_hbm.at[pl.ds(b0, CHUNK_SEQ)], idx_v.at[kb])
    for s in range(CHUNK_SEQ):
        pltpu.async_copy(
            tok_hbm.at[idx_v.at[kb].at[s]],
            rows_v.at[kb].at[s],
            gsem[kb],
        )


def _wait_gather(tok_hbm, idx_v, rows_v, gsem, kb):
    for s in range(CHUNK_SEQ):
        pltpu.make_async_copy(
            tok_hbm.at[idx_v.at[kb].at[s]],
            rows_v.at[kb].at[s],
            gsem[kb],
        ).wait()


def _wait_ocopy(rows_v, out_hbm, osem, kb):
    pltpu.make_async_copy(
        rows_v.at[kb], out_hbm.at[pl.ds(0, CHUNK_SEQ)], osem[kb]
    ).wait()


def _add_pos(rows_v, pos_v, kb):
    @plsc.parallel_loop(0, L, unroll=2)
    def l_body(l):
        pv = [pos_v[l, pl.ds(jj * 16, 16)] for jj in range(NVEC)]
        for s in range(CHUNK_SEQ):
            for jj in range(NVEC):
                sl = pl.ds(jj * 16, 16)
                rows_v[kb, s, l, sl] = rows_v[kb, s, l, sl] + pv[jj]


def _emb_body(x_hbm, tok_hbm, pos_hbm, out_hbm, idx_v, rows_v, pos_v,
              gs0, gs1, gs2, gs3, os0, os1, os2, os3):
    gsem = [gs0, gs1, gs2, gs3]
    osem = [os0, os1, os2, os3]
    wid = lax.axis_index("s") * 2 + lax.axis_index("c")
    base = wid * NCHUNK
    pltpu.sync_copy(pos_hbm, pos_v)

    for c in range(NBUF - 1):
        _issue_chunk(x_hbm, tok_hbm, idx_v, rows_v, gsem, c,
                     (base + c) * CHUNK_SEQ)

    def group_body(g, carry):
        for k in range(NBUF):
            c = g * NBUF + k
            b0 = (base + c) * CHUNK_SEQ
            _wait_gather(tok_hbm, idx_v, rows_v, gsem, k)
            _add_pos(rows_v, pos_v, k)
            pltpu.async_copy(
                rows_v.at[k], out_hbm.at[pl.ds(b0, CHUNK_SEQ)], osem[k]
            )
            kn = (k + NBUF - 1) % NBUF
            cn = c + NBUF - 1

            @pl.when(cn < NCHUNK)
            def _issue_next():
                @pl.when(c >= 1)
                def _drain_prev():
                    _wait_ocopy(rows_v, out_hbm, osem, kn)

                _issue_chunk(x_hbm, tok_hbm, idx_v, rows_v, gsem, kn,
                             (base + cn) * CHUNK_SEQ)

        return carry

    lax.fori_loop(0, NCHUNK // NBUF, group_body, 0)
    for k in range(NBUF):
        _wait_ocopy(rows_v, out_hbm, osem, k)


_emb = functools.partial(
    pl.kernel,
    out_type=jax.ShapeDtypeStruct((B, L, D), jnp.float32),
    mesh=plsc.VectorSubcoreMesh(core_axis_name="c", subcore_axis_name="s"),
    scratch_types=[
        pltpu.VMEM((NBUF, CHUNK_SEQ, L), jnp.int32),
        pltpu.VMEM((NBUF, CHUNK_SEQ, L, D), jnp.float32),
        pltpu.VMEM((L, D), jnp.float32),
    ] + [pltpu.SemaphoreType.DMA] * (2 * NBUF),
    compiler_params=pltpu.CompilerParams(use_tc_tiling_on_sc=True),
)(_emb_body)


def kernel(x, token_table, pos_table):
    return _emb(x.astype(jnp.int32), token_table, pos_table)
